# Initial kernel scaffold; baseline (speedup 1.0000x reference)
import jax
import jax.numpy as jnp
from jax.experimental import pallas as pl


def kernel(x, edge_index, enc_w1, enc_b1, enc_w2, enc_b2, conv_w0, conv_b0, conv_w1, conv_b1, conv_w2, conv_b2, mu_w1, mu_b1, mu_w2, mu_b2, lv_w1, lv_b1, lv_w2, lv_b2):
    raise NotImplementedError("write your pallas kernel here")


# speedup vs baseline: 8.3697x; 8.3697x over previous
import functools

import jax
import jax.numpy as jnp
from jax import lax
from jax.experimental import pallas as pl
from jax.experimental.pallas import tpu as pltpu
from jax.experimental.pallas import tpu_sc as plsc

N = 10000
E = 320000
F_IN = 128
H = 256
HH = 128
D = 128

NC = 2
NS = 16
CH = 128
IDXB = 8
NBLK = 20
K = NBLK * IDXB
E_PAD = NS * K * CH
ACC_ROWS = 10240
ZROWS = ACC_ROWS // NS

RB = 2000
RGRID = N // RB


def _zero_buf2d(buf, rows, cols):
  z = jnp.zeros((16,), jnp.float32)

  def row(i, _):
    def col(j, _):
      buf[i, pl.ds(j * 16, 16)] = z
      return 0
    lax.fori_loop(0, cols // 16, col, 0)
    return 0

  lax.fori_loop(0, rows, row, 0)


def _fill_buf2d(buf, rows, cols, val):
  v = jnp.full((16,), val, jnp.float32)

  def row(i, _):
    def col(j, _):
      buf[i, pl.ds(j * 16, 16)] = v
      return 0
    lax.fori_loop(0, cols // 16, col, 0)
    return 0

  lax.fori_loop(0, rows, row, 0)


def _sc_degree(dst_g):
  mesh = plsc.VectorSubcoreMesh(core_axis_name="c", subcore_axis_name="s")
  KC = K // NC

  @functools.partial(
      pl.kernel,
      out_type=(
          jax.ShapeDtypeStruct((ACC_ROWS, HH), jnp.float32),
          jax.ShapeDtypeStruct((ACC_ROWS, HH), jnp.float32),
      ),
      mesh=mesh,
      scratch_types=[
          pltpu.VMEM((IDXB, CH), jnp.int32),
          pltpu.VMEM((CH, HH), jnp.float32),
          pltpu.VMEM((CH, HH), jnp.float32),
          pltpu.VMEM_SHARED((ACC_ROWS, HH), jnp.float32),
      ],
  )
  def body(dst_hbm, outa_hbm, outb_hbm, idx_dst, ones, bounce, hist):
    cid = lax.axis_index("c")
    sid = lax.axis_index("s")
    _fill_buf2d(ones, CH, HH, 1.0)
    _zero_buf2d(bounce, CH, HH)
    def z(i, _):
      pltpu.sync_copy(bounce, hist.at[pl.ds(sid * ZROWS + i * CH, CH)])
      return 0
    lax.fori_loop(0, ZROWS // CH, z, 0)
    plsc.subcore_barrier()

    def blk_body(blk, _):
      pltpu.sync_copy(
          dst_hbm.at[sid, pl.ds((cid * KC + blk * IDXB), IDXB)], idx_dst)
      for jj in range(IDXB):
        pltpu.sync_copy(ones, hist.at[idx_dst.at[jj]], add=True)
      return 0
    lax.fori_loop(0, KC // IDXB, blk_body, 0)
    plsc.subcore_barrier()

    def wb(out_ref):
      def w(i, _):
        base = sid * ZROWS + i * CH
        pltpu.sync_copy(hist.at[pl.ds(base, CH)], bounce)
        pltpu.sync_copy(bounce, out_ref.at[pl.ds(base, CH)])
        return 0
      lax.fori_loop(0, ZROWS // CH, w, 0)

    @pl.when(cid == 0)
    def _():
      wb(outa_hbm)

    @pl.when(cid == 1)
    def _():
      wb(outb_hbm)

  return body(dst_g)


def _sc_gather_scatter(hs0, hs1, src_g, dst_g):
  mesh = plsc.VectorSubcoreMesh(core_axis_name="c", subcore_axis_name="s")

  @functools.partial(
      pl.kernel,
      out_type=(
          jax.ShapeDtypeStruct((ACC_ROWS, HH), jnp.float32),
          jax.ShapeDtypeStruct((ACC_ROWS, HH), jnp.float32),
      ),
      mesh=mesh,
      scratch_types=[
          pltpu.VMEM((IDXB, CH), jnp.int32),
          pltpu.VMEM((IDXB, CH), jnp.int32),
          pltpu.VMEM((CH, HH), jnp.float32),
          pltpu.VMEM((CH, HH), jnp.float32),
          pltpu.VMEM_SHARED((ACC_ROWS, HH), jnp.float32),
          pltpu.SemaphoreType.DMA,
          pltpu.SemaphoreType.DMA,
      ],
  )
  def body(hs0_hbm, hs1_hbm, src_hbm, dst_hbm, out0_hbm, out1_hbm,
           idx_src, idx_dst, gbufa, gbufb, acc, sema, semb):
    cid = lax.axis_index("c")
    sid = lax.axis_index("s")
    _zero_buf2d(gbufa, CH, HH)
    def z(i, _):
      pltpu.sync_copy(gbufa, acc.at[pl.ds(sid * ZROWS + i * CH, CH)])
      return 0
    lax.fori_loop(0, ZROWS // CH, z, 0)
    plsc.subcore_barrier()

    def run(hs_ref):
      def blk_body(blk, _):
        pltpu.sync_copy(src_hbm.at[sid, pl.ds(blk * IDXB, IDXB)], idx_src)
        pltpu.sync_copy(dst_hbm.at[sid, pl.ds(blk * IDXB, IDXB)], idx_dst)
        pltpu.async_copy(hs_ref.at[idx_src.at[0]], gbufa, sema)
        for jj in range(IDXB):
          if jj % 2 == 0:
            buf, sem, nbuf, nsem = gbufa, sema, gbufb, semb
          else:
            buf, sem, nbuf, nsem = gbufb, semb, gbufa, sema
          if jj + 1 < IDXB:
            pltpu.async_copy(hs_ref.at[idx_src.at[jj + 1]], nbuf, nsem)
          pltpu.make_async_copy(hs_ref.at[idx_src.at[jj]], buf, sem).wait()
          pltpu.sync_copy(buf, acc.at[idx_dst.at[jj]], add=True)
        return 0

      lax.fori_loop(0, NBLK, blk_body, 0)

    @pl.when(cid == 0)
    def _():
      run(hs0_hbm)

    @pl.when(cid == 1)
    def _():
      run(hs1_hbm)

    plsc.subcore_barrier()

    def wb(out_ref):
      def w(i, _):
        base = sid * ZROWS + i * CH
        pltpu.sync_copy(acc.at[pl.ds(base, CH)], gbufa)
        pltpu.sync_copy(gbufa, out_ref.at[pl.ds(base, CH)])
        return 0
      lax.fori_loop(0, ZROWS // CH, w, 0)

    @pl.when(cid == 0)
    def _():
      wb(out0_hbm)

    @pl.when(cid == 1)
    def _():
      wb(out1_hbm)

  return body(hs0, hs1, src_g, dst_g)


def _tc_encoder(x, w1, b1, w2, b2):
  def body(x_ref, w1_ref, b1_ref, w2_ref, b2_ref, h_ref):
    a = jnp.dot(x_ref[...], w1_ref[...], preferred_element_type=jnp.float32)
    a = jax.nn.relu(a + b1_ref[...])
    h_ref[...] = jnp.dot(a, w2_ref[...],
                         preferred_element_type=jnp.float32) + b2_ref[...]

  return pl.pallas_call(
      body,
      grid=(RGRID,),
      in_specs=[
          pl.BlockSpec((RB, F_IN), lambda i: (i, 0)),
          pl.BlockSpec((F_IN, H), lambda i: (0, 0)),
          pl.BlockSpec((1, H), lambda i: (0, 0)),
          pl.BlockSpec((H, H), lambda i: (0, 0)),
          pl.BlockSpec((1, H), lambda i: (0, 0)),
      ],
      out_specs=pl.BlockSpec((RB, H), lambda i: (i, 0)),
      out_shape=jax.ShapeDtypeStruct((N, H), jnp.float32),
  )(x, w1, b1, w2, b2)


def _dis_from_deg(dega_blk, degb_blk):
  return lax.rsqrt(dega_blk[:, 0:1] + degb_blk[:, 0:1] + 1.0)


def _tc_pre(h, dega, degb, w):
  def body(h_ref, dega_ref, degb_ref, w_ref, hs0_ref, hs1_ref):
    dis = _dis_from_deg(dega_ref[...], degb_ref[...])
    hl = jnp.dot(h_ref[...], w_ref[...], preferred_element_type=jnp.float32)
    hs = dis * hl
    hs0_ref[...] = hs[:, :HH]
    hs1_ref[...] = hs[:, HH:]

  return pl.pallas_call(
      body,
      grid=(RGRID,),
      in_specs=[
          pl.BlockSpec((RB, H), lambda i: (i, 0)),
          pl.BlockSpec((RB, HH), lambda i: (i, 0)),
          pl.BlockSpec((RB, HH), lambda i: (i, 0)),
          pl.BlockSpec((H, H), lambda i: (0, 0)),
      ],
      out_specs=[
          pl.BlockSpec((RB, HH), lambda i: (i, 0)),
          pl.BlockSpec((RB, HH), lambda i: (i, 0)),
      ],
      out_shape=[
          jax.ShapeDtypeStruct((N, HH), jnp.float32),
          jax.ShapeDtypeStruct((N, HH), jnp.float32),
      ],
  )(h, dega, degb, w)


def _tc_mid(acc0, acc1, hs0, hs1, dega, degb, b, w_next):
  def body(a0_ref, a1_ref, s0_ref, s1_ref, dega_ref, degb_ref, b_ref, w_ref,
           o0_ref, o1_ref):
    dis = _dis_from_deg(dega_ref[...], degb_ref[...])
    u = jnp.concatenate(
        [a0_ref[...] + s0_ref[...], a1_ref[...] + s1_ref[...]], axis=1)
    hp = jax.nn.relu(dis * u + b_ref[...])
    hs = dis * jnp.dot(hp, w_ref[...], preferred_element_type=jnp.float32)
    o0_ref[...] = hs[:, :HH]
    o1_ref[...] = hs[:, HH:]

  return pl.pallas_call(
      body,
      grid=(RGRID,),
      in_specs=[
          pl.BlockSpec((RB, HH), lambda i: (i, 0)),
          pl.BlockSpec((RB, HH), lambda i: (i, 0)),
          pl.BlockSpec((RB, HH), lambda i: (i, 0)),
          pl.BlockSpec((RB, HH), lambda i: (i, 0)),
          pl.BlockSpec((RB, HH), lambda i: (i, 0)),
          pl.BlockSpec((RB, HH), lambda i: (i, 0)),
          pl.BlockSpec((1, H), lambda i: (0, 0)),
          pl.BlockSpec((H, H), lambda i: (0, 0)),
      ],
      out_specs=[
          pl.BlockSpec((RB, HH), lambda i: (i, 0)),
          pl.BlockSpec((RB, HH), lambda i: (i, 0)),
      ],
      out_shape=[
          jax.ShapeDtypeStruct((N, HH), jnp.float32),
          jax.ShapeDtypeStruct((N, HH), jnp.float32),
      ],
  )(acc0, acc1, hs0, hs1, dega, degb, b, w_next)


def _tc_final(acc0, acc1, hs0, hs1, dega, degb, b,
              mu_w1, mu_b1, mu_w2, mu_b2, lv_w1, lv_b1, lv_w2, lv_b2):
  def body(a0_ref, a1_ref, s0_ref, s1_ref, dega_ref, degb_ref, b_ref,
           mw1_ref, mb1_ref, mw2_ref, mb2_ref,
           lw1_ref, lb1_ref, lw2_ref, lb2_ref,
           mu_ref, lv_ref, sum_ref, max_ref):
    i = pl.program_id(0)
    dis = _dis_from_deg(dega_ref[...], degb_ref[...])
    u = jnp.concatenate(
        [a0_ref[...] + s0_ref[...], a1_ref[...] + s1_ref[...]], axis=1)
    h3 = jax.nn.relu(dis * u + b_ref[...])

    @pl.when(i == 0)
    def _():
      sum_ref[...] = jnp.zeros_like(sum_ref)
      max_ref[...] = jnp.full_like(max_ref, -jnp.inf)

    sum_ref[0:1, :] += jnp.sum(h3, axis=0, keepdims=True)
    max_ref[0:1, :] = jnp.maximum(max_ref[0:1, :],
                                  jnp.max(h3, axis=0, keepdims=True))

    @pl.when(i == RGRID - 1)
    def _():
      g = jnp.concatenate(
          [sum_ref[0:1, :] * (1.0 / N), max_ref[0:1, :]], axis=1)
      gb = jnp.broadcast_to(g, (8, 2 * H))
      tm = jax.nn.relu(
          jnp.dot(gb, mw1_ref[...], preferred_element_type=jnp.float32)
          + mb1_ref[...])
      mu_ref[...] = jnp.dot(
          tm, mw2_ref[...], preferred_element_type=jnp.float32) + mb2_ref[...]
      tl = jax.nn.relu(
          jnp.dot(gb, lw1_ref[...], preferred_element_type=jnp.float32)
          + lb1_ref[...])
      lv_ref[...] = jnp.dot(
          tl, lw2_ref[...], preferred_element_type=jnp.float32) + lb2_ref[...]

  return pl.pallas_call(
      body,
      grid=(RGRID,),
      in_specs=[
          pl.BlockSpec((RB, HH), lambda i: (i, 0)),
          pl.BlockSpec((RB, HH), lambda i: (i, 0)),
          pl.BlockSpec((RB, HH), lambda i: (i, 0)),
          pl.BlockSpec((RB, HH), lambda i: (i, 0)),
          pl.BlockSpec((RB, HH), lambda i: (i, 0)),
          pl.BlockSpec((RB, HH), lambda i: (i, 0)),
          pl.BlockSpec((1, H), lambda i: (0, 0)),
          pl.BlockSpec((2 * H, H), lambda i: (0, 0)),
          pl.BlockSpec((1, H), lambda i: (0, 0)),
          pl.BlockSpec((H, D), lambda i: (0, 0)),
          pl.BlockSpec((1, D), lambda i: (0, 0)),
          pl.BlockSpec((2 * H, H), lambda i: (0, 0)),
          pl.BlockSpec((1, H), lambda i: (0, 0)),
          pl.BlockSpec((H, D), lambda i: (0, 0)),
          pl.BlockSpec((1, D), lambda i: (0, 0)),
      ],
      out_specs=[
          pl.BlockSpec((8, D), lambda i: (0, 0)),
          pl.BlockSpec((8, D), lambda i: (0, 0)),
      ],
      out_shape=[
          jax.ShapeDtypeStruct((8, D), jnp.float32),
          jax.ShapeDtypeStruct((8, D), jnp.float32),
      ],
      scratch_shapes=[
          pltpu.VMEM((8, H), jnp.float32),
          pltpu.VMEM((8, H), jnp.float32),
      ],
  )(acc0, acc1, hs0, hs1, dega, degb, b,
    mu_w1, mu_b1, mu_w2, mu_b2, lv_w1, lv_b1, lv_w2, lv_b2)


def kernel(x, edge_index, enc_w1, enc_b1, enc_w2, enc_b2,
           conv_w0, conv_b0, conv_w1, conv_b1, conv_w2, conv_b2,
           mu_w1, mu_b1, mu_w2, mu_b2, lv_w1, lv_b1, lv_w2, lv_b2):
  src = edge_index[0]
  dst = edge_index[1]
  pad = E_PAD - E
  src_g = jnp.concatenate(
      [src, jnp.zeros((pad,), jnp.int32)]).reshape(NS, K, CH)
  dst_g = jnp.concatenate(
      [dst, jnp.full((pad,), N, jnp.int32)]).reshape(NS, K, CH)

  dega, degb = _sc_degree(dst_g)
  dega, degb = dega[:N], degb[:N]
  h = _tc_encoder(x, enc_w1, enc_b1.reshape(1, H), enc_w2, enc_b2.reshape(1, H))

  hs0, hs1 = _tc_pre(h, dega, degb, conv_w0)
  acc0, acc1 = _sc_gather_scatter(hs0, hs1, src_g, dst_g)
  hs0, hs1 = _tc_mid(acc0[:N], acc1[:N], hs0, hs1, dega, degb,
                     conv_b0.reshape(1, H), conv_w1)
  acc0, acc1 = _sc_gather_scatter(hs0, hs1, src_g, dst_g)
  hs0, hs1 = _tc_mid(acc0[:N], acc1[:N], hs0, hs1, dega, degb,
                     conv_b1.reshape(1, H), conv_w2)
  acc0, acc1 = _sc_gather_scatter(hs0, hs1, src_g, dst_g)
  acc0, acc1 = acc0[:N], acc1[:N]
  mu8, lv8 = _tc_final(acc0, acc1, hs0, hs1, dega, degb, conv_b2.reshape(1, H),
                       mu_w1, mu_b1.reshape(1, H), mu_w2, mu_b2.reshape(1, D),
                       lv_w1, lv_b1.reshape(1, H), lv_w2, lv_b2.reshape(1, D))
  return (mu8[0:1], lv8[0:1])


# speedup vs baseline: 8.7541x; 1.0459x over previous
import functools

import jax
import jax.numpy as jnp
from jax import lax
from jax.experimental import pallas as pl
from jax.experimental.pallas import tpu as pltpu
from jax.experimental.pallas import tpu_sc as plsc

N = 10000
E = 320000
F_IN = 128
H = 256
HH = 128
D = 128

NC = 2
NS = 16
CH = 128
IDXB = 16
NBLK = 10
K = NBLK * IDXB
E_PAD = NS * K * CH
ACC_ROWS = 10240
ZROWS = ACC_ROWS // NS

RB = 2000
RGRID = N // RB


def _zero_buf2d(buf, rows, cols):
  z = jnp.zeros((16,), jnp.float32)

  def row(i, _):
    def col(j, _):
      buf[i, pl.ds(j * 16, 16)] = z
      return 0
    lax.fori_loop(0, cols // 16, col, 0)
    return 0

  lax.fori_loop(0, rows, row, 0)


def _fill_buf2d(buf, rows, cols, val):
  v = jnp.full((16,), val, jnp.float32)

  def row(i, _):
    def col(j, _):
      buf[i, pl.ds(j * 16, 16)] = v
      return 0
    lax.fori_loop(0, cols // 16, col, 0)
    return 0

  lax.fori_loop(0, rows, row, 0)


def _sc_degree(dst_g):
  mesh = plsc.VectorSubcoreMesh(core_axis_name="c", subcore_axis_name="s")
  KC = K // NC

  @functools.partial(
      pl.kernel,
      out_type=(
          jax.ShapeDtypeStruct((ACC_ROWS, HH), jnp.float32),
          jax.ShapeDtypeStruct((ACC_ROWS, HH), jnp.float32),
      ),
      mesh=mesh,
      scratch_types=[
          pltpu.VMEM((IDXB, CH), jnp.int32),
          pltpu.VMEM((CH, HH), jnp.float32),
          pltpu.VMEM((CH, HH), jnp.float32),
          pltpu.VMEM_SHARED((ACC_ROWS, HH), jnp.float32),
      ],
  )
  def body(dst_hbm, outa_hbm, outb_hbm, idx_dst, ones, bounce, hist):
    cid = lax.axis_index("c")
    sid = lax.axis_index("s")
    _fill_buf2d(ones, CH, HH, 1.0)
    _zero_buf2d(bounce, CH, HH)
    def z(i, _):
      pltpu.sync_copy(bounce, hist.at[pl.ds(sid * ZROWS + i * CH, CH)])
      return 0
    lax.fori_loop(0, ZROWS // CH, z, 0)
    plsc.subcore_barrier()

    def blk_body(blk, _):
      pltpu.sync_copy(
          dst_hbm.at[sid, pl.ds((cid * KC + blk * IDXB), IDXB)], idx_dst)
      for jj in range(IDXB):
        pltpu.sync_copy(ones, hist.at[idx_dst.at[jj]], add=True)
      return 0
    lax.fori_loop(0, KC // IDXB, blk_body, 0)
    plsc.subcore_barrier()

    def wb(out_ref):
      def w(i, _):
        base = sid * ZROWS + i * CH
        pltpu.sync_copy(hist.at[pl.ds(base, CH)], bounce)
        pltpu.sync_copy(bounce, out_ref.at[pl.ds(base, CH)])
        return 0
      lax.fori_loop(0, ZROWS // CH, w, 0)

    @pl.when(cid == 0)
    def _():
      wb(outa_hbm)

    @pl.when(cid == 1)
    def _():
      wb(outb_hbm)

  return body(dst_g)


def _sc_gather_scatter(hs0, hs1, src_g, dst_g):
  mesh = plsc.VectorSubcoreMesh(core_axis_name="c", subcore_axis_name="s")

  @functools.partial(
      pl.kernel,
      out_type=(
          jax.ShapeDtypeStruct((ACC_ROWS, HH), jnp.float32),
          jax.ShapeDtypeStruct((ACC_ROWS, HH), jnp.float32),
      ),
      mesh=mesh,
      scratch_types=[
          pltpu.VMEM((2, IDXB, CH), jnp.int32),
          pltpu.VMEM((2, IDXB, CH), jnp.int32),
          pltpu.VMEM((CH, HH), jnp.float32),
          pltpu.VMEM((CH, HH), jnp.float32),
          pltpu.VMEM_SHARED((ACC_ROWS, HH), jnp.float32),
          pltpu.SemaphoreType.DMA,
          pltpu.SemaphoreType.DMA,
          pltpu.SemaphoreType.DMA,
          pltpu.SemaphoreType.DMA,
      ],
  )
  def body(hs0_hbm, hs1_hbm, src_hbm, dst_hbm, out0_hbm, out1_hbm,
           idx_src, idx_dst, gbufa, gbufb, acc, gsa, gsb, ssa, ssb):
    cid = lax.axis_index("c")
    sid = lax.axis_index("s")
    _zero_buf2d(gbufa, CH, HH)
    def z(i, _):
      pltpu.sync_copy(gbufa, acc.at[pl.ds(sid * ZROWS + i * CH, CH)])
      return 0
    lax.fori_loop(0, ZROWS // CH, z, 0)
    plsc.subcore_barrier()

    def run(hs_ref):
      def stage(b, par):
        pltpu.sync_copy(src_hbm.at[sid, pl.ds(b * IDXB, IDXB)],
                        idx_src.at[par])
        pltpu.sync_copy(dst_hbm.at[sid, pl.ds(b * IDXB, IDXB)],
                        idx_dst.at[par])

      stage(0, 0)
      pltpu.async_copy(hs_ref.at[idx_src.at[0, 0]], gbufa, gsa)

      def blk_body(b, _):
        p = lax.rem(b, 2)
        pn = 1 - p

        @pl.when(b + 1 < NBLK)
        def _():
          stage(b + 1, pn)

        for jj in range(IDXB):
          if jj % 2 == 0:
            X, gsX, ssX = gbufa, gsa, ssa
            Y, gsY, ssY = gbufb, gsb, ssb
          else:
            X, gsX, ssX = gbufb, gsb, ssb
            Y, gsY, ssY = gbufa, gsa, ssa
          if jj == 0:
            @pl.when(b > 0)
            def _():
              pltpu.make_async_copy(Y, acc.at[idx_dst.at[p, jj]], ssY).wait()
            pltpu.async_copy(hs_ref.at[idx_src.at[p, jj + 1]], Y, gsY)
          else:
            pltpu.make_async_copy(Y, acc.at[idx_dst.at[p, jj]], ssY).wait()
            if jj + 1 < IDXB:
              pltpu.async_copy(hs_ref.at[idx_src.at[p, jj + 1]], Y, gsY)
            else:
              @pl.when(b + 1 < NBLK)
              def _():
                pltpu.async_copy(hs_ref.at[idx_src.at[pn, 0]], Y, gsY)
          pltpu.make_async_copy(hs_ref.at[idx_src.at[p, jj]], X, gsX).wait()
          pltpu.async_copy(X, acc.at[idx_dst.at[p, jj]], ssX, add=True)
        return 0

      lax.fori_loop(0, NBLK, blk_body, 0)
      pl_last = (NBLK - 1) % 2
      pltpu.make_async_copy(
          gbufb, acc.at[idx_dst.at[pl_last, IDXB - 1]], ssb).wait()

    @pl.when(cid == 0)
    def _():
      run(hs0_hbm)

    @pl.when(cid == 1)
    def _():
      run(hs1_hbm)

    plsc.subcore_barrier()

    def wb(out_ref):
      def w(i, _):
        base = sid * ZROWS + i * CH
        pltpu.sync_copy(acc.at[pl.ds(base, CH)], gbufa)
        pltpu.sync_copy(gbufa, out_ref.at[pl.ds(base, CH)])
        return 0
      lax.fori_loop(0, ZROWS // CH, w, 0)

    @pl.when(cid == 0)
    def _():
      wb(out0_hbm)

    @pl.when(cid == 1)
    def _():
      wb(out1_hbm)

  return body(hs0, hs1, src_g, dst_g)


def _tc_encoder(x, w1, b1, w2, b2):
  def body(x_ref, w1_ref, b1_ref, w2_ref, b2_ref, h_ref):
    a = jnp.dot(x_ref[...], w1_ref[...], preferred_element_type=jnp.float32)
    a = jax.nn.relu(a + b1_ref[...])
    h_ref[...] = jnp.dot(a, w2_ref[...],
                         preferred_element_type=jnp.float32) + b2_ref[...]

  return pl.pallas_call(
      body,
      grid=(RGRID,),
      in_specs=[
          pl.BlockSpec((RB, F_IN), lambda i: (i, 0)),
          pl.BlockSpec((F_IN, H), lambda i: (0, 0)),
          pl.BlockSpec((1, H), lambda i: (0, 0)),
          pl.BlockSpec((H, H), lambda i: (0, 0)),
          pl.BlockSpec((1, H), lambda i: (0, 0)),
      ],
      out_specs=pl.BlockSpec((RB, H), lambda i: (i, 0)),
      out_shape=jax.ShapeDtypeStruct((N, H), jnp.float32),
  )(x, w1, b1, w2, b2)


def _dis_from_deg(dega_blk, degb_blk):
  return lax.rsqrt(dega_blk[:, 0:1] + degb_blk[:, 0:1] + 1.0)


def _tc_pre(h, dega, degb, w):
  def body(h_ref, dega_ref, degb_ref, w_ref, hs0_ref, hs1_ref):
    dis = _dis_from_deg(dega_ref[...], degb_ref[...])
    hl = jnp.dot(h_ref[...], w_ref[...], preferred_element_type=jnp.float32)
    hs = dis * hl
    hs0_ref[...] = hs[:, :HH]
    hs1_ref[...] = hs[:, HH:]

  return pl.pallas_call(
      body,
      grid=(RGRID,),
      in_specs=[
          pl.BlockSpec((RB, H), lambda i: (i, 0)),
          pl.BlockSpec((RB, HH), lambda i: (i, 0)),
          pl.BlockSpec((RB, HH), lambda i: (i, 0)),
          pl.BlockSpec((H, H), lambda i: (0, 0)),
      ],
      out_specs=[
          pl.BlockSpec((RB, HH), lambda i: (i, 0)),
          pl.BlockSpec((RB, HH), lambda i: (i, 0)),
      ],
      out_shape=[
          jax.ShapeDtypeStruct((N, HH), jnp.float32),
          jax.ShapeDtypeStruct((N, HH), jnp.float32),
      ],
  )(h, dega, degb, w)


def _tc_mid(acc0, acc1, hs0, hs1, dega, degb, b, w_next):
  def body(a0_ref, a1_ref, s0_ref, s1_ref, dega_ref, degb_ref, b_ref, w_ref,
           o0_ref, o1_ref):
    dis = _dis_from_deg(dega_ref[...], degb_ref[...])
    u = jnp.concatenate(
        [a0_ref[...] + s0_ref[...], a1_ref[...] + s1_ref[...]], axis=1)
    hp = jax.nn.relu(dis * u + b_ref[...])
    hs = dis * jnp.dot(hp, w_ref[...], preferred_element_type=jnp.float32)
    o0_ref[...] = hs[:, :HH]
    o1_ref[...] = hs[:, HH:]

  return pl.pallas_call(
      body,
      grid=(RGRID,),
      in_specs=[
          pl.BlockSpec((RB, HH), lambda i: (i, 0)),
          pl.BlockSpec((RB, HH), lambda i: (i, 0)),
          pl.BlockSpec((RB, HH), lambda i: (i, 0)),
          pl.BlockSpec((RB, HH), lambda i: (i, 0)),
          pl.BlockSpec((RB, HH), lambda i: (i, 0)),
          pl.BlockSpec((RB, HH), lambda i: (i, 0)),
          pl.BlockSpec((1, H), lambda i: (0, 0)),
          pl.BlockSpec((H, H), lambda i: (0, 0)),
      ],
      out_specs=[
          pl.BlockSpec((RB, HH), lambda i: (i, 0)),
          pl.BlockSpec((RB, HH), lambda i: (i, 0)),
      ],
      out_shape=[
          jax.ShapeDtypeStruct((N, HH), jnp.float32),
          jax.ShapeDtypeStruct((N, HH), jnp.float32),
      ],
  )(acc0, acc1, hs0, hs1, dega, degb, b, w_next)


def _tc_final(acc0, acc1, hs0, hs1, dega, degb, b,
              mu_w1, mu_b1, mu_w2, mu_b2, lv_w1, lv_b1, lv_w2, lv_b2):
  def body(a0_ref, a1_ref, s0_ref, s1_ref, dega_ref, degb_ref, b_ref,
           mw1_ref, mb1_ref, mw2_ref, mb2_ref,
           lw1_ref, lb1_ref, lw2_ref, lb2_ref,
           mu_ref, lv_ref, sum_ref, max_ref):
    i = pl.program_id(0)
    dis = _dis_from_deg(dega_ref[...], degb_ref[...])
    u = jnp.concatenate(
        [a0_ref[...] + s0_ref[...], a1_ref[...] + s1_ref[...]], axis=1)
    h3 = jax.nn.relu(dis * u + b_ref[...])

    @pl.when(i == 0)
    def _():
      sum_ref[...] = jnp.zeros_like(sum_ref)
      max_ref[...] = jnp.full_like(max_ref, -jnp.inf)

    sum_ref[0:1, :] += jnp.sum(h3, axis=0, keepdims=True)
    max_ref[0:1, :] = jnp.maximum(max_ref[0:1, :],
                                  jnp.max(h3, axis=0, keepdims=True))

    @pl.when(i == RGRID - 1)
    def _():
      g = jnp.concatenate(
          [sum_ref[0:1, :] * (1.0 / N), max_ref[0:1, :]], axis=1)
      gb = jnp.broadcast_to(g, (8, 2 * H))
      tm = jax.nn.relu(
          jnp.dot(gb, mw1_ref[...], preferred_element_type=jnp.float32)
          + mb1_ref[...])
      mu_ref[...] = jnp.dot(
          tm, mw2_ref[...], preferred_element_type=jnp.float32) + mb2_ref[...]
      tl = jax.nn.relu(
          jnp.dot(gb, lw1_ref[...], preferred_element_type=jnp.float32)
          + lb1_ref[...])
      lv_ref[...] = jnp.dot(
          tl, lw2_ref[...], preferred_element_type=jnp.float32) + lb2_ref[...]

  return pl.pallas_call(
      body,
      grid=(RGRID,),
      in_specs=[
          pl.BlockSpec((RB, HH), lambda i: (i, 0)),
          pl.BlockSpec((RB, HH), lambda i: (i, 0)),
          pl.BlockSpec((RB, HH), lambda i: (i, 0)),
          pl.BlockSpec((RB, HH), lambda i: (i, 0)),
          pl.BlockSpec((RB, HH), lambda i: (i, 0)),
          pl.BlockSpec((RB, HH), lambda i: (i, 0)),
          pl.BlockSpec((1, H), lambda i: (0, 0)),
          pl.BlockSpec((2 * H, H), lambda i: (0, 0)),
          pl.BlockSpec((1, H), lambda i: (0, 0)),
          pl.BlockSpec((H, D), lambda i: (0, 0)),
          pl.BlockSpec((1, D), lambda i: (0, 0)),
          pl.BlockSpec((2 * H, H), lambda i: (0, 0)),
          pl.BlockSpec((1, H), lambda i: (0, 0)),
          pl.BlockSpec((H, D), lambda i: (0, 0)),
          pl.BlockSpec((1, D), lambda i: (0, 0)),
      ],
      out_specs=[
          pl.BlockSpec((8, D), lambda i: (0, 0)),
          pl.BlockSpec((8, D), lambda i: (0, 0)),
      ],
      out_shape=[
          jax.ShapeDtypeStruct((8, D), jnp.float32),
          jax.ShapeDtypeStruct((8, D), jnp.float32),
      ],
      scratch_shapes=[
          pltpu.VMEM((8, H), jnp.float32),
          pltpu.VMEM((8, H), jnp.float32),
      ],
  )(acc0, acc1, hs0, hs1, dega, degb, b,
    mu_w1, mu_b1, mu_w2, mu_b2, lv_w1, lv_b1, lv_w2, lv_b2)


def kernel(x, edge_index, enc_w1, enc_b1, enc_w2, enc_b2,
           conv_w0, conv_b0, conv_w1, conv_b1, conv_w2, conv_b2,
           mu_w1, mu_b1, mu_w2, mu_b2, lv_w1, lv_b1, lv_w2, lv_b2):
  src = edge_index[0]
  dst = edge_index[1]
  pad = E_PAD - E
  src_g = jnp.concatenate(
      [src, jnp.zeros((pad,), jnp.int32)]).reshape(NS, K, CH)
  dst_g = jnp.concatenate(
      [dst, jnp.full((pad,), N, jnp.int32)]).reshape(NS, K, CH)

  dega, degb = _sc_degree(dst_g)
  dega, degb = dega[:N], degb[:N]
  h = _tc_encoder(x, enc_w1, enc_b1.reshape(1, H), enc_w2, enc_b2.reshape(1, H))

  hs0, hs1 = _tc_pre(h, dega, degb, conv_w0)
  acc0, acc1 = _sc_gather_scatter(hs0, hs1, src_g, dst_g)
  hs0, hs1 = _tc_mid(acc0[:N], acc1[:N], hs0, hs1, dega, degb,
                     conv_b0.reshape(1, H), conv_w1)
  acc0, acc1 = _sc_gather_scatter(hs0, hs1, src_g, dst_g)
  hs0, hs1 = _tc_mid(acc0[:N], acc1[:N], hs0, hs1, dega, degb,
                     conv_b1.reshape(1, H), conv_w2)
  acc0, acc1 = _sc_gather_scatter(hs0, hs1, src_g, dst_g)
  acc0, acc1 = acc0[:N], acc1[:N]
  mu8, lv8 = _tc_final(acc0, acc1, hs0, hs1, dega, degb, conv_b2.reshape(1, H),
                       mu_w1, mu_b1.reshape(1, H), mu_w2, mu_b2.reshape(1, D),
                       lv_w1, lv_b1.reshape(1, H), lv_w2, lv_b2.reshape(1, D))
  return (mu8[0:1], lv8[0:1])


# speedup vs baseline: 9.0313x; 1.0317x over previous
import functools

import jax
import jax.numpy as jnp
from jax import lax
from jax.experimental import pallas as pl
from jax.experimental.pallas import tpu as pltpu
from jax.experimental.pallas import tpu_sc as plsc

N = 10000
E = 320000
F_IN = 128
H = 256
HH = 128
D = 128

NC = 2
NS = 16
CH = 128
IDXB = 16
NBLK = 10
K = NBLK * IDXB
E_PAD = NS * K * CH
ACC_ROWS = 10240
ZROWS = ACC_ROWS // NS

RB = 2000
RGRID = N // RB


def _zero_buf2d(buf, rows, cols):
  z = jnp.zeros((16,), jnp.float32)

  def row(i, _):
    def col(j, _):
      buf[i, pl.ds(j * 16, 16)] = z
      return 0
    lax.fori_loop(0, cols // 16, col, 0)
    return 0

  lax.fori_loop(0, rows, row, 0)


def _fill_buf2d(buf, rows, cols, val):
  v = jnp.full((16,), val, jnp.float32)

  def row(i, _):
    def col(j, _):
      buf[i, pl.ds(j * 16, 16)] = v
      return 0
    lax.fori_loop(0, cols // 16, col, 0)
    return 0

  lax.fori_loop(0, rows, row, 0)


def _sc_degree(dst_g):
  mesh = plsc.VectorSubcoreMesh(core_axis_name="c", subcore_axis_name="s")
  KC = K // NC

  @functools.partial(
      pl.kernel,
      out_type=(
          jax.ShapeDtypeStruct((ACC_ROWS, HH), jnp.float32),
          jax.ShapeDtypeStruct((ACC_ROWS, HH), jnp.float32),
      ),
      mesh=mesh,
      scratch_types=[
          pltpu.VMEM((IDXB, CH), jnp.int32),
          pltpu.VMEM((CH, HH), jnp.float32),
          pltpu.VMEM((CH, HH), jnp.float32),
          pltpu.VMEM_SHARED((ACC_ROWS, HH), jnp.float32),
      ],
  )
  def body(dst_hbm, outa_hbm, outb_hbm, idx_dst, ones, bounce, hist):
    cid = lax.axis_index("c")
    sid = lax.axis_index("s")
    _fill_buf2d(ones, CH, HH, 1.0)
    _zero_buf2d(bounce, CH, HH)
    def z(i, _):
      pltpu.sync_copy(bounce, hist.at[pl.ds(sid * ZROWS + i * CH, CH)])
      return 0
    lax.fori_loop(0, ZROWS // CH, z, 0)
    plsc.subcore_barrier()

    def blk_body(blk, _):
      pltpu.sync_copy(
          dst_hbm.at[sid, pl.ds((cid * KC + blk * IDXB), IDXB)], idx_dst)
      for jj in range(IDXB):
        pltpu.sync_copy(ones, hist.at[idx_dst.at[jj]], add=True)
      return 0
    lax.fori_loop(0, KC // IDXB, blk_body, 0)
    plsc.subcore_barrier()

    def wb(out_ref):
      def w(i, _):
        base = sid * ZROWS + i * CH
        pltpu.sync_copy(hist.at[pl.ds(base, CH)], bounce)
        pltpu.sync_copy(bounce, out_ref.at[pl.ds(base, CH)])
        return 0
      lax.fori_loop(0, ZROWS // CH, w, 0)

    @pl.when(cid == 0)
    def _():
      wb(outa_hbm)

    @pl.when(cid == 1)
    def _():
      wb(outb_hbm)

  return body(dst_g)


def _sc_gather_scatter(hs0, hs1, src_g, dst_g):
  mesh = plsc.VectorSubcoreMesh(core_axis_name="c", subcore_axis_name="s")

  @functools.partial(
      pl.kernel,
      out_type=(
          jax.ShapeDtypeStruct((ACC_ROWS, HH), jnp.float32),
          jax.ShapeDtypeStruct((ACC_ROWS, HH), jnp.float32),
      ),
      mesh=mesh,
      scratch_types=[
          pltpu.VMEM((2, IDXB, CH), jnp.int32),
          pltpu.VMEM((2, IDXB, CH), jnp.int32),
          pltpu.VMEM((CH, HH), jnp.float32),
          pltpu.VMEM((CH, HH), jnp.float32),
          pltpu.VMEM_SHARED((ACC_ROWS, HH), jnp.float32),
          pltpu.SemaphoreType.DMA,
          pltpu.SemaphoreType.DMA,
          pltpu.SemaphoreType.DMA,
          pltpu.SemaphoreType.DMA,
      ],
  )
  def body(hs0_hbm, hs1_hbm, src_hbm, dst_hbm, out0_hbm, out1_hbm,
           idx_src, idx_dst, gbufa, gbufb, acc, gsa, gsb, ssa, ssb):
    cid = lax.axis_index("c")
    sid = lax.axis_index("s")
    _zero_buf2d(gbufa, CH, HH)
    def z(i, _):
      pltpu.sync_copy(gbufa, acc.at[pl.ds(sid * ZROWS + i * CH, CH)])
      return 0
    lax.fori_loop(0, ZROWS // CH, z, 0)
    plsc.subcore_barrier()

    def run(hs_ref):
      def stage(b, par):
        pltpu.sync_copy(src_hbm.at[sid, pl.ds(b * IDXB, IDXB)],
                        idx_src.at[par])
        pltpu.sync_copy(dst_hbm.at[sid, pl.ds(b * IDXB, IDXB)],
                        idx_dst.at[par])

      stage(0, 0)
      pltpu.async_copy(hs_ref.at[idx_src.at[0, 0]], gbufa, gsa)

      def blk_body(b, _):
        p = lax.rem(b, 2)
        pn = 1 - p

        @pl.when(b + 1 < NBLK)
        def _():
          stage(b + 1, pn)

        for jj in range(IDXB):
          if jj % 2 == 0:
            X, gsX, ssX = gbufa, gsa, ssa
            Y, gsY, ssY = gbufb, gsb, ssb
          else:
            X, gsX, ssX = gbufb, gsb, ssb
            Y, gsY, ssY = gbufa, gsa, ssa
          if jj == 0:
            @pl.when(b > 0)
            def _():
              pltpu.make_async_copy(Y, acc.at[idx_dst.at[p, jj]], ssY).wait()
            pltpu.async_copy(hs_ref.at[idx_src.at[p, jj + 1]], Y, gsY)
          else:
            pltpu.make_async_copy(Y, acc.at[idx_dst.at[p, jj]], ssY).wait()
            if jj + 1 < IDXB:
              pltpu.async_copy(hs_ref.at[idx_src.at[p, jj + 1]], Y, gsY)
            else:
              @pl.when(b + 1 < NBLK)
              def _():
                pltpu.async_copy(hs_ref.at[idx_src.at[pn, 0]], Y, gsY)
          pltpu.make_async_copy(hs_ref.at[idx_src.at[p, jj]], X, gsX).wait()
          pltpu.async_copy(X, acc.at[idx_dst.at[p, jj]], ssX, add=True)
        return 0

      lax.fori_loop(0, NBLK, blk_body, 0)
      pl_last = (NBLK - 1) % 2
      pltpu.make_async_copy(
          gbufb, acc.at[idx_dst.at[pl_last, IDXB - 1]], ssb).wait()

    @pl.when(cid == 0)
    def _():
      run(hs0_hbm)

    @pl.when(cid == 1)
    def _():
      run(hs1_hbm)

    plsc.subcore_barrier()

    def wb(out_ref):
      def w(i, _):
        base = sid * ZROWS + i * CH
        pltpu.sync_copy(acc.at[pl.ds(base, CH)], gbufa)
        pltpu.sync_copy(gbufa, out_ref.at[pl.ds(base, CH)])
        return 0
      lax.fori_loop(0, ZROWS // CH, w, 0)

    @pl.when(cid == 0)
    def _():
      wb(out0_hbm)

    @pl.when(cid == 1)
    def _():
      wb(out1_hbm)

  return body(hs0, hs1, src_g, dst_g)


def _tc_encoder(x, w1, b1, w2, b2):
  def body(x_ref, w1_ref, b1_ref, w2_ref, b2_ref, h_ref):
    a = jnp.dot(x_ref[...], w1_ref[...], preferred_element_type=jnp.float32)
    a = jax.nn.relu(a + b1_ref[...])
    h_ref[...] = jnp.dot(a, w2_ref[...],
                         preferred_element_type=jnp.float32) + b2_ref[...]

  return pl.pallas_call(
      body,
      grid=(RGRID,),
      in_specs=[
          pl.BlockSpec((RB, F_IN), lambda i: (i, 0)),
          pl.BlockSpec((F_IN, H), lambda i: (0, 0)),
          pl.BlockSpec((1, H), lambda i: (0, 0)),
          pl.BlockSpec((H, H), lambda i: (0, 0)),
          pl.BlockSpec((1, H), lambda i: (0, 0)),
      ],
      out_specs=pl.BlockSpec((RB, H), lambda i: (i, 0)),
      out_shape=jax.ShapeDtypeStruct((N, H), jnp.float32),
  )(x, w1, b1, w2, b2)


def _dis_from_deg(dega_blk, degb_blk):
  return lax.rsqrt(dega_blk[:, 0:1] + degb_blk[:, 0:1] + 1.0)


def _tc_pre(h, dega, degb, w):
  def body(h_ref, dega_ref, degb_ref, w_ref, hs0_ref, hs1_ref):
    dis = _dis_from_deg(dega_ref[...], degb_ref[...])
    hl = jnp.dot(h_ref[...], w_ref[...], preferred_element_type=jnp.float32)
    hs = dis * hl
    hs0_ref[...] = hs[:, :HH]
    hs1_ref[...] = hs[:, HH:]

  return pl.pallas_call(
      body,
      grid=(RGRID,),
      in_specs=[
          pl.BlockSpec((RB, H), lambda i: (i, 0)),
          pl.BlockSpec((RB, HH), lambda i: (i, 0)),
          pl.BlockSpec((RB, HH), lambda i: (i, 0)),
          pl.BlockSpec((H, H), lambda i: (0, 0)),
      ],
      out_specs=[
          pl.BlockSpec((RB, HH), lambda i: (i, 0)),
          pl.BlockSpec((RB, HH), lambda i: (i, 0)),
      ],
      out_shape=[
          jax.ShapeDtypeStruct((N, HH), jnp.float32),
          jax.ShapeDtypeStruct((N, HH), jnp.float32),
      ],
  )(h, dega, degb, w)


def _tc_mid(acc0, acc1, hs0, hs1, dega, degb, b, w_next):
  def body(a0_ref, a1_ref, s0_ref, s1_ref, dega_ref, degb_ref, b_ref, w_ref,
           o0_ref, o1_ref):
    dis = _dis_from_deg(dega_ref[...], degb_ref[...])
    u = jnp.concatenate(
        [a0_ref[...] + s0_ref[...], a1_ref[...] + s1_ref[...]], axis=1)
    hp = jax.nn.relu(dis * u + b_ref[...])
    hs = dis * jnp.dot(hp, w_ref[...], preferred_element_type=jnp.float32)
    o0_ref[...] = hs[:, :HH]
    o1_ref[...] = hs[:, HH:]

  return pl.pallas_call(
      body,
      grid=(RGRID,),
      in_specs=[
          pl.BlockSpec((RB, HH), lambda i: (i, 0)),
          pl.BlockSpec((RB, HH), lambda i: (i, 0)),
          pl.BlockSpec((RB, HH), lambda i: (i, 0)),
          pl.BlockSpec((RB, HH), lambda i: (i, 0)),
          pl.BlockSpec((RB, HH), lambda i: (i, 0)),
          pl.BlockSpec((RB, HH), lambda i: (i, 0)),
          pl.BlockSpec((1, H), lambda i: (0, 0)),
          pl.BlockSpec((H, H), lambda i: (0, 0)),
      ],
      out_specs=[
          pl.BlockSpec((RB, HH), lambda i: (i, 0)),
          pl.BlockSpec((RB, HH), lambda i: (i, 0)),
      ],
      out_shape=[
          jax.ShapeDtypeStruct((N, HH), jnp.float32),
          jax.ShapeDtypeStruct((N, HH), jnp.float32),
      ],
  )(acc0, acc1, hs0, hs1, dega, degb, b, w_next)


def _tc_final(acc0, acc1, hs0, hs1, dega, degb, b,
              mu_w1, mu_b1, mu_w2, mu_b2, lv_w1, lv_b1, lv_w2, lv_b2):
  def body(a0_ref, a1_ref, s0_ref, s1_ref, dega_ref, degb_ref, b_ref,
           mw1_ref, mb1_ref, mw2_ref, mb2_ref,
           lw1_ref, lb1_ref, lw2_ref, lb2_ref,
           mu_ref, lv_ref, sum_ref, max_ref):
    i = pl.program_id(0)
    dis = _dis_from_deg(dega_ref[...], degb_ref[...])
    u = jnp.concatenate(
        [a0_ref[...] + s0_ref[...], a1_ref[...] + s1_ref[...]], axis=1)
    h3 = jax.nn.relu(dis * u + b_ref[...])

    @pl.when(i == 0)
    def _():
      sum_ref[...] = jnp.zeros_like(sum_ref)
      max_ref[...] = jnp.full_like(max_ref, -jnp.inf)

    sum_ref[0:1, :] += jnp.sum(h3, axis=0, keepdims=True)
    max_ref[0:1, :] = jnp.maximum(max_ref[0:1, :],
                                  jnp.max(h3, axis=0, keepdims=True))

    @pl.when(i == RGRID - 1)
    def _():
      g = jnp.concatenate(
          [sum_ref[0:1, :] * (1.0 / N), max_ref[0:1, :]], axis=1)
      gb = jnp.broadcast_to(g, (8, 2 * H))
      tm = jax.nn.relu(
          jnp.dot(gb, mw1_ref[...], preferred_element_type=jnp.float32)
          + mb1_ref[...])
      mu_ref[...] = jnp.dot(
          tm, mw2_ref[...], preferred_element_type=jnp.float32) + mb2_ref[...]
      tl = jax.nn.relu(
          jnp.dot(gb, lw1_ref[...], preferred_element_type=jnp.float32)
          + lb1_ref[...])
      lv_ref[...] = jnp.dot(
          tl, lw2_ref[...], preferred_element_type=jnp.float32) + lb2_ref[...]

  return pl.pallas_call(
      body,
      grid=(RGRID,),
      in_specs=[
          pl.BlockSpec((RB, HH), lambda i: (i, 0)),
          pl.BlockSpec((RB, HH), lambda i: (i, 0)),
          pl.BlockSpec((RB, HH), lambda i: (i, 0)),
          pl.BlockSpec((RB, HH), lambda i: (i, 0)),
          pl.BlockSpec((RB, HH), lambda i: (i, 0)),
          pl.BlockSpec((RB, HH), lambda i: (i, 0)),
          pl.BlockSpec((1, H), lambda i: (0, 0)),
          pl.BlockSpec((2 * H, H), lambda i: (0, 0)),
          pl.BlockSpec((1, H), lambda i: (0, 0)),
          pl.BlockSpec((H, D), lambda i: (0, 0)),
          pl.BlockSpec((1, D), lambda i: (0, 0)),
          pl.BlockSpec((2 * H, H), lambda i: (0, 0)),
          pl.BlockSpec((1, H), lambda i: (0, 0)),
          pl.BlockSpec((H, D), lambda i: (0, 0)),
          pl.BlockSpec((1, D), lambda i: (0, 0)),
      ],
      out_specs=[
          pl.BlockSpec((8, D), lambda i: (0, 0)),
          pl.BlockSpec((8, D), lambda i: (0, 0)),
      ],
      out_shape=[
          jax.ShapeDtypeStruct((8, D), jnp.float32),
          jax.ShapeDtypeStruct((8, D), jnp.float32),
      ],
      scratch_shapes=[
          pltpu.VMEM((8, H), jnp.float32),
          pltpu.VMEM((8, H), jnp.float32),
      ],
  )(acc0, acc1, hs0, hs1, dega, degb, b,
    mu_w1, mu_b1, mu_w2, mu_b2, lv_w1, lv_b1, lv_w2, lv_b2)


def kernel(x, edge_index, enc_w1, enc_b1, enc_w2, enc_b2,
           conv_w0, conv_b0, conv_w1, conv_b1, conv_w2, conv_b2,
           mu_w1, mu_b1, mu_w2, mu_b2, lv_w1, lv_b1, lv_w2, lv_b2):
  src = edge_index[0]
  dst = edge_index[1]
  pad = E_PAD - E
  pad_dst = N + jnp.arange(pad, dtype=jnp.int32) % (ACC_ROWS - N)
  src_g = jnp.concatenate(
      [src, jnp.zeros((pad,), jnp.int32)]).reshape(NS, K, CH)
  dst_g = jnp.concatenate([dst, pad_dst]).reshape(NS, K, CH)

  dega, degb = _sc_degree(dst_g)
  dega, degb = dega[:N], degb[:N]
  h = _tc_encoder(x, enc_w1, enc_b1.reshape(1, H), enc_w2, enc_b2.reshape(1, H))

  hs0, hs1 = _tc_pre(h, dega, degb, conv_w0)
  acc0, acc1 = _sc_gather_scatter(hs0, hs1, src_g, dst_g)
  hs0, hs1 = _tc_mid(acc0[:N], acc1[:N], hs0, hs1, dega, degb,
                     conv_b0.reshape(1, H), conv_w1)
  acc0, acc1 = _sc_gather_scatter(hs0, hs1, src_g, dst_g)
  hs0, hs1 = _tc_mid(acc0[:N], acc1[:N], hs0, hs1, dega, degb,
                     conv_b1.reshape(1, H), conv_w2)
  acc0, acc1 = _sc_gather_scatter(hs0, hs1, src_g, dst_g)
  acc0, acc1 = acc0[:N], acc1[:N]
  mu8, lv8 = _tc_final(acc0, acc1, hs0, hs1, dega, degb, conv_b2.reshape(1, H),
                       mu_w1, mu_b1.reshape(1, H), mu_w2, mu_b2.reshape(1, D),
                       lv_w1, lv_b1.reshape(1, H), lv_w2, lv_b2.reshape(1, D))
  return (mu8[0:1], lv8[0:1])
